# baseline (speedup 1.0000x reference)
import jax
import jax.numpy as jnp
from jax.experimental import pallas as pl
from jax.experimental.pallas import tpu as pltpu

_B, _G, _K, _D = 16384, 4, 512, 64
_BETA = 0.25
_BT = 256


def _vq_body(x_ref, p_ref, recon_ref, loss_ref, dists_ref):
    acc = jnp.float32(0.0)
    for g in range(_G):
        x = x_ref[:, g * _D:(g + 1) * _D]
        p = p_ref[g]
        pp = jnp.sum(p * p, axis=1)
        xp = jax.lax.dot_general(
            x, p, (((1,), (1,)), ((), ())),
            preferred_element_type=jnp.float32)
        xxm = jax.lax.dot_general(
            x * x, jnp.ones((_D, _K), jnp.float32),
            (((1,), (0,)), ((), ())),
            preferred_element_type=jnp.float32)
        d = xxm - 2.0 * xp + pp[None, :]
        dists_ref[g] = d
        lanes = 128
        iota_l = jax.lax.broadcasted_iota(jnp.int32, (_BT, lanes), 1).astype(jnp.float32)
        m = d[:, :lanes]
        ki = iota_l
        for c in range(1, _K // lanes):
            dc = d[:, c * lanes:(c + 1) * lanes]
            upd = dc < m
            m = jnp.where(upd, dc, m)
            ki = jnp.where(upd, iota_l + c * lanes, ki)
        min_d = jnp.min(m, axis=1, keepdims=True)
        ind = jnp.min(jnp.where(m == min_d, ki, float(_K)), axis=1)
        iota_k = jax.lax.broadcasted_iota(jnp.int32, (_BT, _K), 1).astype(jnp.float32)
        one_hot = (iota_k == ind[:, None]).astype(jnp.float32)
        q = jax.lax.dot_general(
            one_hot, p, (((1,), (0,)), ((), ())),
            preferred_element_type=jnp.float32)
        recon_ref[:, g * _D:(g + 1) * _D] = q
        acc += jnp.sum(min_d)
    loss_ref[0, 0, 0] = acc


@jax.jit
def kernel(latents, protos):
    n_tiles = _B // _BT
    x2d = latents.reshape(_B, _G * _D)
    recon, loss_parts, dists = pl.pallas_call(
        _vq_body,
        grid=(n_tiles,),
        in_specs=[
            pl.BlockSpec((_BT, _G * _D), lambda i: (i, 0)),
            pl.BlockSpec((_G, _K, _D), lambda i: (0, 0, 0)),
        ],
        out_specs=[
            pl.BlockSpec((_BT, _G * _D), lambda i: (i, 0)),
            pl.BlockSpec((1, 1, 1), lambda i: (i, 0, 0), memory_space=pltpu.SMEM),
            pl.BlockSpec((_G, _BT, _K), lambda i: (0, i, 0)),
        ],
        out_shape=[
            jax.ShapeDtypeStruct((_B, _G * _D), jnp.float32),
            jax.ShapeDtypeStruct((n_tiles, 1, 1), jnp.float32),
            jax.ShapeDtypeStruct((_G, _B, _K), jnp.float32),
        ],
    )(x2d, protos)
    scale = jnp.float32((1.0 + _BETA) / (_G * _B * _D))
    vq_loss = jnp.sum(loss_parts) * scale
    return recon.reshape(_B, _G, _D), vq_loss, dists


# speedup vs baseline: 1.3664x; 1.3664x over previous
import jax
import jax.numpy as jnp
from jax.experimental import pallas as pl
from jax.experimental.pallas import tpu as pltpu

_B, _G, _K, _D = 16384, 4, 512, 64
_BETA = 0.25
_BT = 1024


def _vq_body(x_ref, p_ref, recon_ref, loss_ref, dists_ref):
    acc = jnp.float32(0.0)
    for g in range(_G):
        x = x_ref[:, g * _D:(g + 1) * _D]
        p = p_ref[g]
        pp = jnp.sum(p * p, axis=1)
        xp = jax.lax.dot_general(
            x, p, (((1,), (1,)), ((), ())),
            preferred_element_type=jnp.float32)
        xxm = jax.lax.dot_general(
            x * x, jnp.ones((_D, _K), jnp.float32),
            (((1,), (0,)), ((), ())),
            preferred_element_type=jnp.float32)
        d = xxm - 2.0 * xp + pp[None, :]
        dists_ref[g] = d
        lanes = 128
        iota_l = jax.lax.broadcasted_iota(jnp.int32, (_BT, lanes), 1).astype(jnp.float32)
        m = d[:, :lanes]
        ki = iota_l
        for c in range(1, _K // lanes):
            dc = d[:, c * lanes:(c + 1) * lanes]
            upd = dc < m
            m = jnp.where(upd, dc, m)
            ki = jnp.where(upd, iota_l + c * lanes, ki)
        min_d = jnp.min(m, axis=1, keepdims=True)
        ind = jnp.min(jnp.where(m == min_d, ki, float(_K)), axis=1)
        iota_k = jax.lax.broadcasted_iota(jnp.int32, (_BT, _K), 1).astype(jnp.float32)
        one_hot = (iota_k == ind[:, None]).astype(jnp.float32)
        q = jax.lax.dot_general(
            one_hot, p, (((1,), (0,)), ((), ())),
            preferred_element_type=jnp.float32)
        recon_ref[:, g * _D:(g + 1) * _D] = q
        acc += jnp.sum(min_d)
    loss_ref[0, 0, 0] = acc


@jax.jit
def kernel(latents, protos):
    n_tiles = _B // _BT
    x2d = latents.reshape(_B, _G * _D)
    recon, loss_parts, dists = pl.pallas_call(
        _vq_body,
        grid=(n_tiles,),
        in_specs=[
            pl.BlockSpec((_BT, _G * _D), lambda i: (i, 0)),
            pl.BlockSpec((_G, _K, _D), lambda i: (0, 0, 0)),
        ],
        out_specs=[
            pl.BlockSpec((_BT, _G * _D), lambda i: (i, 0)),
            pl.BlockSpec((1, 1, 1), lambda i: (i, 0, 0), memory_space=pltpu.SMEM),
            pl.BlockSpec((_G, _BT, _K), lambda i: (0, i, 0)),
        ],
        out_shape=[
            jax.ShapeDtypeStruct((_B, _G * _D), jnp.float32),
            jax.ShapeDtypeStruct((n_tiles, 1, 1), jnp.float32),
            jax.ShapeDtypeStruct((_G, _B, _K), jnp.float32),
        ],
    )(x2d, protos)
    scale = jnp.float32((1.0 + _BETA) / (_G * _B * _D))
    vq_loss = jnp.sum(loss_parts) * scale
    return recon.reshape(_B, _G, _D), vq_loss, dists
